# Initial kernel scaffold; baseline (speedup 1.0000x reference)
import jax
import jax.numpy as jnp
from jax.experimental import pallas as pl


def kernel(x, e, e_subgraph, edge_index, edge_index_sub, W_node, b_node, W_edge, b_edge, A, bA, B, bB, C, bC, U, bU, V, bV, W_pred, b_pred):
    raise NotImplementedError("write your pallas kernel here")


# speedup vs baseline: 1.0476x; 1.0476x over previous
import functools

import jax
import jax.numpy as jnp
from jax import lax
from jax.experimental import pallas as pl
from jax.experimental.pallas import tpu as pltpu
from jax.experimental.pallas import tpu_sc as plsc

N = 10000
E = 320000
D_NODE = 128
D_EDGE = 16
H = 128
HH = 64
LN = 16
NC = 2
NS = 16
EPT = E // NS
CH = 32
NCHUNK = EPT // CH
NP = 10112
NPT = NP // NS
EPW = E // (NC * NS)
CH2 = 400
NCHUNK2 = EPW // CH2
BE = 4000

_f32 = jnp.float32


def _node_first_body(x_ref, wn_ref, bn_ref, a_ref, ba_ref, b_ref, bb_ref,
                     v_ref, bv_ref, u_ref, bu_ref, ah_o, bvc_o, uh_o):
    h = jnp.dot(x_ref[...], wn_ref[...], preferred_element_type=_f32) + bn_ref[...]
    ah = jnp.dot(h, a_ref[...], preferred_element_type=_f32) + ba_ref[...]
    bh = jnp.dot(h, b_ref[...], preferred_element_type=_f32) + bb_ref[...]
    vh = jnp.dot(h, v_ref[...], preferred_element_type=_f32) + bv_ref[...]
    uh = jnp.dot(h, u_ref[...], preferred_element_type=_f32) + bu_ref[...]
    ah_o[...] = ah
    bvc_o[...] = jnp.concatenate(
        [bh[:, :HH], vh[:, :HH], bh[:, HH:], vh[:, HH:]], axis=1)
    uh_o[...] = uh


def _node_next_body(uh_ref, nd_ref, a_ref, ba_ref, b_ref, bb_ref,
                    v_ref, bv_ref, u_ref, bu_ref, ah_o, bvc_o, uh_o):
    nd = nd_ref[...]
    agg = jnp.concatenate(
        [nd[:N, :HH] / (nd[:N, HH:] + 1e-6),
         nd[NP:NP + N, :HH] / (nd[NP:NP + N, HH:] + 1e-6)], axis=1)
    h = jnp.maximum(uh_ref[...] + agg, 0.0)
    ah = jnp.dot(h, a_ref[...], preferred_element_type=_f32) + ba_ref[...]
    bh = jnp.dot(h, b_ref[...], preferred_element_type=_f32) + bb_ref[...]
    vh = jnp.dot(h, v_ref[...], preferred_element_type=_f32) + bv_ref[...]
    uh = jnp.dot(h, u_ref[...], preferred_element_type=_f32) + bu_ref[...]
    ah_o[...] = ah
    bvc_o[...] = jnp.concatenate(
        [bh[:, :HH], vh[:, :HH], bh[:, HH:], vh[:, HH:]], axis=1)
    uh_o[...] = uh


def _node_last_body(uh_ref, nd_ref, w12_ref, bvec_ref, p_o):
    nd = nd_ref[...]
    agg = jnp.concatenate(
        [nd[:N, :HH] / (nd[:N, HH:] + 1e-6),
         nd[NP:NP + N, :HH] / (nd[NP:NP + N, HH:] + 1e-6)], axis=1)
    h = jnp.maximum(uh_ref[...] + agg, 0.0)
    p_o[...] = jnp.dot(h, w12_ref[...], preferred_element_type=_f32) + bvec_ref[...]


def _node_first(x, wn, bn, a, ba, b, bb, v, bv, u, bu):
    return pl.pallas_call(
        _node_first_body,
        out_shape=[
            jax.ShapeDtypeStruct((N, H), _f32),
            jax.ShapeDtypeStruct((N, 2 * H), _f32),
            jax.ShapeDtypeStruct((N, H), _f32),
        ],
    )(x, wn, bn, a, ba, b, bb, v, bv, u, bu)


def _node_next(uh, nd, a, ba, b, bb, v, bv, u, bu):
    return pl.pallas_call(
        _node_next_body,
        out_shape=[
            jax.ShapeDtypeStruct((N, H), _f32),
            jax.ShapeDtypeStruct((N, 2 * H), _f32),
            jax.ShapeDtypeStruct((N, H), _f32),
        ],
    )(uh, nd, a, ba, b, bb, v, bv, u, bu)


def _node_last(uh, nd, w12, bvec):
    return pl.pallas_call(
        _node_last_body,
        out_shape=jax.ShapeDtypeStruct((N, 2), _f32),
    )(uh, nd, w12, bvec)


def _edge_first_body(e_ref, we_ref, be_ref, cs_ref, bcs_ref, out_ref):
    ee = jnp.dot(e_ref[...], we_ref[...], preferred_element_type=_f32) + be_ref[...]
    out_ref[0] = (jnp.dot(ee, cs_ref[0], preferred_element_type=_f32)
                  + bcs_ref[0])


def _edge_first(e, we, be, cs, bcs):
    grid = (2, E // BE)
    return pl.pallas_call(
        _edge_first_body,
        grid=grid,
        in_specs=[
            pl.BlockSpec((BE, D_EDGE), lambda c, i: (i, 0)),
            pl.BlockSpec((D_EDGE, H), lambda c, i: (0, 0)),
            pl.BlockSpec((1, H), lambda c, i: (0, 0)),
            pl.BlockSpec((1, H, HH), lambda c, i: (c, 0, 0)),
            pl.BlockSpec((1, 1, HH), lambda c, i: (c, 0, 0)),
        ],
        out_specs=pl.BlockSpec((1, BE, HH), lambda c, i: (c, i, 0)),
        out_shape=jax.ShapeDtypeStruct((2, E, HH), _f32),
    )(e, we, be, cs, bcs)


def _edge_next_body(ee_ref, cs_ref, bcs_ref, out_ref):
    cblk = cs_ref[0]
    out_ref[0] = (jnp.dot(ee_ref[0], cblk[:HH], preferred_element_type=_f32)
                  + jnp.dot(ee_ref[1], cblk[HH:], preferred_element_type=_f32)
                  + bcs_ref[0])


def _edge_next(ee, cs, bcs):
    grid = (2, E // BE)
    return pl.pallas_call(
        _edge_next_body,
        grid=grid,
        in_specs=[
            pl.BlockSpec((2, BE, HH), lambda c, i: (0, i, 0)),
            pl.BlockSpec((1, H, HH), lambda c, i: (c, 0, 0)),
            pl.BlockSpec((1, 1, HH), lambda c, i: (c, 0, 0)),
        ],
        out_specs=pl.BlockSpec((1, BE, HH), lambda c, i: (c, i, 0)),
        out_shape=jax.ShapeDtypeStruct((2, E, HH), _f32),
    )(ee, cs, bcs)


def _make_edge_pass(write_ee):
    mesh = plsc.VectorSubcoreMesh(
        core_axis_name="c", subcore_axis_name="s", num_cores=NC, num_subcores=NS)
    if write_ee:
        out_type = [jax.ShapeDtypeStruct((2 * NP, H), _f32),
                    jax.ShapeDtypeStruct((2 * E, HH), _f32)]
    else:
        out_type = jax.ShapeDtypeStruct((2 * NP, H), _f32)
    scratch = [
        pltpu.VMEM((CH,), jnp.int32),
        pltpu.VMEM((CH,), jnp.int32),
        pltpu.VMEM((CH, H), _f32),
        pltpu.VMEM((CH, H), _f32),
        pltpu.VMEM((CH, HH), _f32),
        pltpu.VMEM((CH, HH), _f32),
        pltpu.VMEM((CH, H), _f32),
        pltpu.VMEM_SHARED((NP, H), _f32),
        pltpu.SemaphoreType.DMA,
        pltpu.SemaphoreType.DMA,
    ]

    @functools.partial(pl.kernel, out_type=out_type, mesh=mesh,
                       scratch_types=scratch)
    def edge_pass(dst_hbm, src_hbm, ah_hbm, bvc_hbm, ce_hbm, zero_hbm, *rest):
        if write_ee:
            nd_out, ee_out = rest[0], rest[1]
            (dsti_v, srci2_v, a_v, bv_v, ce_v, ee_v, ps_v, nd_sp,
             sem_a, sem_b) = rest[2:]
        else:
            nd_out = rest[0]
            ee_out = None
            (dsti_v, srci2_v, a_v, bv_v, ce_v, ee_v, ps_v, nd_sp,
             sem_a, sem_b) = rest[1:]
        c = lax.axis_index("c")
        s = lax.axis_index("s")
        r0 = s * NPT
        pltpu.sync_copy(zero_hbm.at[pl.ds(r0, NPT)], nd_sp.at[pl.ds(r0, NPT)])
        plsc.subcore_barrier()
        ebase = s * EPT

        def chunk(g, carry):
            base = ebase + g * CH
            pltpu.sync_copy(dst_hbm.at[pl.ds(base, CH)], dsti_v)
            pltpu.sync_copy(src_hbm.at[pl.ds(base, CH)], srci2_v)
            for kk in range(CH // LN):
                sl = pl.ds(kk * LN, LN)
                srci2_v[sl] = srci2_v[sl] * 2 + c
            pltpu.async_copy(ah_hbm.at[dsti_v], a_v, sem_a).wait()
            pltpu.async_copy(bvc_hbm.at[srci2_v], bv_v, sem_b).wait()
            pltpu.sync_copy(ce_hbm.at[pl.ds(c * E + base, CH)], ce_v)
            abase = c * HH

            def edge(i, ecarry):
                for r in range(HH // LN):
                    sl = pl.ds(r * LN, LN)
                    sv = pl.ds(HH + r * LN, LN)
                    en = (a_v[i, pl.ds(abase + r * LN, LN)]
                          + bv_v[i, sl] + ce_v[i, sl])
                    sg = 1.0 / (1.0 + jnp.exp(-en))
                    ps_v[i, sl] = sg * bv_v[i, sv]
                    ps_v[i, sv] = sg
                    if write_ee:
                        ee_v[i, sl] = jnp.maximum(en, 0.0)
                return ecarry

            lax.fori_loop(0, CH, edge, 0)
            if write_ee:
                pltpu.sync_copy(ee_v, ee_out.at[pl.ds(c * E + base, CH)])
            pltpu.sync_copy(ps_v, nd_sp.at[dsti_v], add=True)
            return carry

        lax.fori_loop(0, NCHUNK, chunk, 0)
        plsc.subcore_barrier()
        pltpu.sync_copy(nd_sp.at[pl.ds(r0, NPT)],
                        nd_out.at[pl.ds(c * NP + r0, NPT)])

    return edge_pass


_edge_pass_l0 = _make_edge_pass(True)
_edge_pass_l1 = _make_edge_pass(False)


def _make_score():
    mesh = plsc.VectorSubcoreMesh(
        core_axis_name="c", subcore_axis_name="s", num_cores=NC, num_subcores=NS)
    scratch = [
        pltpu.VMEM((N,), _f32),
        pltpu.VMEM((N,), _f32),
        pltpu.VMEM((CH2,), jnp.int32),
        pltpu.VMEM((CH2,), jnp.int32),
        pltpu.VMEM((CH2,), _f32),
    ]

    @functools.partial(pl.kernel,
                       out_type=jax.ShapeDtypeStruct((E,), _f32),
                       mesh=mesh, scratch_types=scratch,
                       compiler_params=pltpu.CompilerParams(
                           needs_layout_passes=False))
    def score(p1_hbm, p2_hbm, ssrc_hbm, sdst_hbm, out_hbm,
              p1_v, p2_v, si_v, di_v, o_v):
        c = lax.axis_index("c")
        s = lax.axis_index("s")
        wid = s * NC + c
        pltpu.sync_copy(p1_hbm, p1_v)
        pltpu.sync_copy(p2_hbm, p2_v)
        wbase = wid * EPW

        def chunk(j, carry):
            base = wbase + j * CH2
            pltpu.sync_copy(ssrc_hbm.at[pl.ds(base, CH2)], si_v)
            pltpu.sync_copy(sdst_hbm.at[pl.ds(base, CH2)], di_v)

            def vec(kk, vcarry):
                sl = pl.ds(kk * LN, LN)
                g1 = plsc.load_gather(p1_v, [si_v[sl]])
                g2 = plsc.load_gather(p2_v, [di_v[sl]])
                o_v[sl] = g1 + g2
                return vcarry

            lax.fori_loop(0, CH2 // LN, vec, 0)
            pltpu.sync_copy(o_v, out_hbm.at[pl.ds(base, CH2)])
            return carry

        lax.fori_loop(0, NCHUNK2, chunk, 0)

    return score


_score_pass = _make_score()


def kernel(x, e, e_subgraph, edge_index, edge_index_sub, W_node, b_node,
           W_edge, b_edge, A, bA, B, bB, C, bC, U, bU, V, bV, W_pred, b_pred):
    del e_subgraph
    dst = edge_index[1]
    src = edge_index[0]
    ssrc = edge_index_sub[0]
    sdst = edge_index_sub[1]
    zeros_nd = jnp.zeros((NP, H), _f32)

    bn = b_node.reshape(1, H)
    be = b_edge.reshape(1, H)
    cs = [jnp.moveaxis(C[i].reshape(H, 2, HH), 1, 0) for i in range(2)]
    bcs = [bC[i].reshape(2, 1, HH) for i in range(2)]
    w12 = jnp.stack([W_pred[:H, 0], W_pred[H:, 0]], axis=1)
    bvec = jnp.concatenate([b_pred, jnp.zeros((1,), _f32)]).reshape(1, 2)

    ah0, bvc0, uh0 = _node_first(
        x, W_node, bn, A[0], bA[0].reshape(1, H), B[0], bB[0].reshape(1, H),
        V[0], bV[0].reshape(1, H), U[0], bU[0].reshape(1, H))
    ce0 = _edge_first(e, W_edge, be, cs[0], bcs[0])
    nd0, ee1 = _edge_pass_l0(
        dst, src, ah0, bvc0.reshape(2 * N, H),
        ce0.reshape(2 * E, HH), zeros_nd)

    ah1, bvc1, uh1 = _node_next(
        uh0, nd0, A[1], bA[1].reshape(1, H), B[1], bB[1].reshape(1, H),
        V[1], bV[1].reshape(1, H), U[1], bU[1].reshape(1, H))
    ce1 = _edge_next(ee1.reshape(2, E, HH), cs[1], bcs[1])
    nd1 = _edge_pass_l1(
        dst, src, ah1, bvc1.reshape(2 * N, H),
        ce1.reshape(2 * E, HH), zeros_nd)

    p12 = _node_last(uh1, nd1, w12, bvec)
    scores = _score_pass(p12[:, 0], p12[:, 1], ssrc, sdst)
    return scores.reshape(E, 1)


# speedup vs baseline: 1.3603x; 1.2985x over previous
import functools

import jax
import jax.numpy as jnp
from jax import lax
from jax.experimental import pallas as pl
from jax.experimental.pallas import tpu as pltpu
from jax.experimental.pallas import tpu_sc as plsc

N = 10000
E = 320000
D_NODE = 128
D_EDGE = 16
H = 128
HH = 64
LN = 16
NC = 2
NS = 16
EPT = E // NS
CH = 32
NCHUNK = EPT // CH
NP = 10112
NPT = NP // NS
EPW = E // (NC * NS)
CH2 = 400
NCHUNK2 = EPW // CH2
BE = 4000

_f32 = jnp.float32


def _node_first_body(x_ref, wn_ref, bn_ref, a_ref, ba_ref, b_ref, bb_ref,
                     v_ref, bv_ref, u_ref, bu_ref, t_o, uh_o):
    h = jnp.dot(x_ref[...], wn_ref[...], preferred_element_type=_f32) + bn_ref[...]
    ah = jnp.dot(h, a_ref[...], preferred_element_type=_f32) + ba_ref[...]
    bh = jnp.dot(h, b_ref[...], preferred_element_type=_f32) + bb_ref[...]
    vh = jnp.dot(h, v_ref[...], preferred_element_type=_f32) + bv_ref[...]
    uh = jnp.dot(h, u_ref[...], preferred_element_type=_f32) + bu_ref[...]
    t_o[0 * N:1 * N] = ah
    t_o[1 * N:2 * N] = jnp.concatenate([bh[:, :HH], vh[:, :HH]], axis=1)
    t_o[2 * N:3 * N] = ah
    t_o[3 * N:4 * N] = jnp.concatenate([bh[:, HH:], vh[:, HH:]], axis=1)
    uh_o[...] = uh


def _node_next_body(uh_ref, nd_ref, a_ref, ba_ref, b_ref, bb_ref,
                    v_ref, bv_ref, u_ref, bu_ref, t_o, uh_o):
    nd = nd_ref[...]
    agg = jnp.concatenate(
        [nd[:N, :HH] / (nd[:N, HH:] + 1e-6),
         nd[NP:NP + N, :HH] / (nd[NP:NP + N, HH:] + 1e-6)], axis=1)
    h = jnp.maximum(uh_ref[...] + agg, 0.0)
    ah = jnp.dot(h, a_ref[...], preferred_element_type=_f32) + ba_ref[...]
    bh = jnp.dot(h, b_ref[...], preferred_element_type=_f32) + bb_ref[...]
    vh = jnp.dot(h, v_ref[...], preferred_element_type=_f32) + bv_ref[...]
    uh = jnp.dot(h, u_ref[...], preferred_element_type=_f32) + bu_ref[...]
    t_o[0 * N:1 * N] = ah
    t_o[1 * N:2 * N] = jnp.concatenate([bh[:, :HH], vh[:, :HH]], axis=1)
    t_o[2 * N:3 * N] = ah
    t_o[3 * N:4 * N] = jnp.concatenate([bh[:, HH:], vh[:, HH:]], axis=1)
    uh_o[...] = uh


def _node_last_body(uh_ref, nd_ref, w12_ref, bvec_ref, p_o):
    nd = nd_ref[...]
    agg = jnp.concatenate(
        [nd[:N, :HH] / (nd[:N, HH:] + 1e-6),
         nd[NP:NP + N, :HH] / (nd[NP:NP + N, HH:] + 1e-6)], axis=1)
    h = jnp.maximum(uh_ref[...] + agg, 0.0)
    p_o[...] = jnp.dot(h, w12_ref[...], preferred_element_type=_f32) + bvec_ref[...]


def _node_first(x, wn, bn, a, ba, b, bb, v, bv, u, bu):
    return pl.pallas_call(
        _node_first_body,
        out_shape=[
            jax.ShapeDtypeStruct((4 * N, H), _f32),
            jax.ShapeDtypeStruct((N, H), _f32),
        ],
    )(x, wn, bn, a, ba, b, bb, v, bv, u, bu)


def _node_next(uh, nd, a, ba, b, bb, v, bv, u, bu):
    return pl.pallas_call(
        _node_next_body,
        out_shape=[
            jax.ShapeDtypeStruct((4 * N, H), _f32),
            jax.ShapeDtypeStruct((N, H), _f32),
        ],
    )(uh, nd, a, ba, b, bb, v, bv, u, bu)


def _node_last(uh, nd, w12, bvec):
    return pl.pallas_call(
        _node_last_body,
        out_shape=jax.ShapeDtypeStruct((N, 2), _f32),
    )(uh, nd, w12, bvec)


def _edge_first_body(e_ref, we_ref, be_ref, cs_ref, bcs_ref, out_ref):
    ee = jnp.dot(e_ref[...], we_ref[...], preferred_element_type=_f32) + be_ref[...]
    out_ref[0] = (jnp.dot(ee, cs_ref[0], preferred_element_type=_f32)
                  + bcs_ref[0])


def _edge_first(e, we, be, cs, bcs):
    grid = (2, E // BE)
    return pl.pallas_call(
        _edge_first_body,
        grid=grid,
        in_specs=[
            pl.BlockSpec((BE, D_EDGE), lambda c, i: (i, 0)),
            pl.BlockSpec((D_EDGE, H), lambda c, i: (0, 0)),
            pl.BlockSpec((1, H), lambda c, i: (0, 0)),
            pl.BlockSpec((1, H, HH), lambda c, i: (c, 0, 0)),
            pl.BlockSpec((1, 1, HH), lambda c, i: (c, 0, 0)),
        ],
        out_specs=pl.BlockSpec((1, BE, HH), lambda c, i: (c, i, 0)),
        out_shape=jax.ShapeDtypeStruct((2, E, HH), _f32),
    )(e, we, be, cs, bcs)


def _edge_next_body(ee_ref, cs_ref, bcs_ref, out_ref):
    cblk = cs_ref[0]
    out_ref[0] = (jnp.dot(ee_ref[0], cblk[:HH], preferred_element_type=_f32)
                  + jnp.dot(ee_ref[1], cblk[HH:], preferred_element_type=_f32)
                  + bcs_ref[0])


def _edge_next(ee, cs, bcs):
    grid = (2, E // BE)
    return pl.pallas_call(
        _edge_next_body,
        grid=grid,
        in_specs=[
            pl.BlockSpec((2, BE, HH), lambda c, i: (0, i, 0)),
            pl.BlockSpec((1, H, HH), lambda c, i: (c, 0, 0)),
            pl.BlockSpec((1, 1, HH), lambda c, i: (c, 0, 0)),
        ],
        out_specs=pl.BlockSpec((1, BE, HH), lambda c, i: (c, i, 0)),
        out_shape=jax.ShapeDtypeStruct((2, E, HH), _f32),
    )(ee, cs, bcs)


def _make_edge_pass(write_ee):
    mesh = plsc.VectorSubcoreMesh(
        core_axis_name="c", subcore_axis_name="s", num_cores=NC, num_subcores=NS)
    if write_ee:
        out_type = [jax.ShapeDtypeStruct((2 * NP, H), _f32),
                    jax.ShapeDtypeStruct((2 * E, HH), _f32)]
    else:
        out_type = jax.ShapeDtypeStruct((2 * NP, H), _f32)
    scratch = [
        pltpu.VMEM((CH,), jnp.int32),
        pltpu.VMEM((CH,), jnp.int32),
        pltpu.VMEM((CH,), jnp.int32),
        pltpu.VMEM((CH,), jnp.int32),
        pltpu.VMEM((CH,), jnp.int32),
        pltpu.VMEM((CH, H), _f32),
        pltpu.VMEM((CH, H), _f32),
        pltpu.VMEM((CH, HH), _f32),
        pltpu.VMEM((CH, HH), _f32),
        pltpu.VMEM((CH, H), _f32),
        pltpu.VMEM_SHARED((NP, H), _f32),
        pltpu.SemaphoreType.DMA,
        pltpu.SemaphoreType.DMA,
        pltpu.SemaphoreType.DMA,
    ]

    @functools.partial(pl.kernel, out_type=out_type, mesh=mesh,
                       scratch_types=scratch)
    def edge_pass(dst_hbm, src_hbm, t_hbm, ce_hbm, *rest):
        if write_ee:
            nd_out, ee_out = rest[0], rest[1]
            rest = rest[2:]
        else:
            nd_out = rest[0]
            ee_out = None
            rest = rest[1:]
        (dsti_v, srci_v, adst_v, bsrc_v, dsc_v, a_v, bv_v, ce_v, ee_v,
         ps_v, nd_sp, sem_i, sem_g, sem_w) = rest
        c = lax.axis_index("c")
        s = lax.axis_index("s")
        r0 = s * NPT
        def _z(i, carry):
            for kk in range(H // LN):
                ps_v[i, pl.ds(kk * LN, LN)] = jnp.zeros((LN,), _f32)
            return carry

        lax.fori_loop(0, CH, _z, 0)

        def _zcopy(k, carry):
            pltpu.sync_copy(ps_v, nd_sp.at[pl.ds(r0 + k * CH, CH)])
            return carry

        lax.fori_loop(0, NPT // CH, _zcopy, 0)
        pltpu.sync_copy(ps_v.at[pl.ds(0, NPT - (NPT // CH) * CH)],
                        nd_sp.at[pl.ds(r0 + (NPT // CH) * CH,
                                       NPT - (NPT // CH) * CH)])
        plsc.subcore_barrier()

        ebase = s * EPT
        cebase = c * E + ebase
        coff = c * (2 * N)

        def issue_idx(g):
            pltpu.async_copy(dst_hbm.at[pl.ds(ebase + g * CH, CH)],
                             dsti_v, sem_i)
            pltpu.async_copy(src_hbm.at[pl.ds(ebase + g * CH, CH)],
                             srci_v, sem_i)

        def wait_idx():
            pltpu.make_async_copy(dst_hbm.at[pl.ds(ebase, CH)],
                                  dsti_v, sem_i).wait()
            pltpu.make_async_copy(src_hbm.at[pl.ds(ebase, CH)],
                                  srci_v, sem_i).wait()

        def transform():
            for kk in range(CH // LN):
                sl = pl.ds(kk * LN, LN)
                d = dsti_v[sl]
                adst_v[sl] = d + coff
                dsc_v[sl] = d
                bsrc_v[sl] = srci_v[sl] + (coff + N)

        def issue_loads(g):
            pltpu.async_copy(t_hbm.at[adst_v], a_v, sem_g)
            pltpu.async_copy(t_hbm.at[bsrc_v], bv_v, sem_g)
            pltpu.async_copy(ce_hbm.at[pl.ds(cebase + g * CH, CH)],
                             ce_v, sem_g)

        def wait_loads():
            pltpu.make_async_copy(t_hbm.at[adst_v], a_v, sem_g).wait()
            pltpu.make_async_copy(t_hbm.at[bsrc_v], bv_v, sem_g).wait()
            pltpu.make_async_copy(ce_hbm.at[pl.ds(cebase, CH)], ce_v,
                                  sem_g).wait()

        def issue_writes(g):
            if write_ee:
                pltpu.async_copy(ee_v, ee_out.at[pl.ds(cebase + g * CH, CH)],
                                 sem_w)
            pltpu.async_copy(ps_v, nd_sp.at[dsc_v], sem_w, add=True)

        def wait_writes():
            if write_ee:
                pltpu.make_async_copy(ee_v, ee_out.at[pl.ds(cebase, CH)],
                                      sem_w).wait()
            pltpu.make_async_copy(ps_v, nd_sp.at[dsc_v], sem_w).wait()

        abase = c * HH

        def compute():
            def edge(i, ecarry):
                for r in range(HH // LN):
                    sl = pl.ds(r * LN, LN)
                    sv = pl.ds(HH + r * LN, LN)
                    en = (a_v[i, pl.ds(abase + r * LN, LN)]
                          + bv_v[i, sl] + ce_v[i, sl])
                    sg = 1.0 / (1.0 + jnp.exp(-en))
                    ps_v[i, sl] = sg * bv_v[i, sv]
                    ps_v[i, sv] = sg
                    if write_ee:
                        ee_v[i, sl] = jnp.maximum(en, 0.0)
                return ecarry

            lax.fori_loop(0, CH, edge, 0)

        def chunk(g, carry):
            d1 = pltpu.async_copy(dst_hbm.at[pl.ds(ebase + g * CH, CH)],
                                  dsti_v, sem_i)
            d2 = pltpu.async_copy(src_hbm.at[pl.ds(ebase + g * CH, CH)],
                                  srci_v, sem_i)
            d1.wait()
            d2.wait()
            transform()
            g1 = pltpu.async_copy(t_hbm.at[adst_v], a_v, sem_g)
            g2 = pltpu.async_copy(t_hbm.at[bsrc_v], bv_v, sem_g)
            g3 = pltpu.async_copy(ce_hbm.at[pl.ds(cebase + g * CH, CH)],
                                  ce_v, sem_g)
            g1.wait()
            g2.wait()
            g3.wait()
            compute()
            if write_ee:
                pltpu.sync_copy(ee_v, ee_out.at[pl.ds(cebase + g * CH, CH)])
            pltpu.sync_copy(ps_v, nd_sp.at[dsc_v], add=True)
            return carry

        lax.fori_loop(0, NCHUNK, chunk, 0)
        plsc.subcore_barrier()
        pltpu.sync_copy(nd_sp.at[pl.ds(r0, NPT)],
                        nd_out.at[pl.ds(c * NP + r0, NPT)])

    return edge_pass


_edge_pass_l0 = _make_edge_pass(True)
_edge_pass_l1 = _make_edge_pass(False)


def _make_score():
    mesh = plsc.VectorSubcoreMesh(
        core_axis_name="c", subcore_axis_name="s", num_cores=NC, num_subcores=NS)
    scratch = [
        pltpu.VMEM((N,), _f32),
        pltpu.VMEM((N,), _f32),
        pltpu.VMEM((CH2,), jnp.int32),
        pltpu.VMEM((CH2,), jnp.int32),
        pltpu.VMEM((CH2,), _f32),
    ]

    @functools.partial(pl.kernel,
                       out_type=jax.ShapeDtypeStruct((E,), _f32),
                       mesh=mesh, scratch_types=scratch,
                       compiler_params=pltpu.CompilerParams(
                           needs_layout_passes=False))
    def score(p1_hbm, p2_hbm, ssrc_hbm, sdst_hbm, out_hbm,
              p1_v, p2_v, si_v, di_v, o_v):
        c = lax.axis_index("c")
        s = lax.axis_index("s")
        wid = s * NC + c
        pltpu.sync_copy(p1_hbm, p1_v)
        pltpu.sync_copy(p2_hbm, p2_v)
        wbase = wid * EPW

        def chunk(j, carry):
            base = wbase + j * CH2
            pltpu.sync_copy(ssrc_hbm.at[pl.ds(base, CH2)], si_v)
            pltpu.sync_copy(sdst_hbm.at[pl.ds(base, CH2)], di_v)

            def vec(kk, vcarry):
                sl = pl.ds(kk * LN, LN)
                g1 = plsc.load_gather(p1_v, [si_v[sl]])
                g2 = plsc.load_gather(p2_v, [di_v[sl]])
                o_v[sl] = g1 + g2
                return vcarry

            lax.fori_loop(0, CH2 // LN, vec, 0)
            pltpu.sync_copy(o_v, out_hbm.at[pl.ds(base, CH2)])
            return carry

        lax.fori_loop(0, NCHUNK2, chunk, 0)

    return score


_score_pass = _make_score()


def kernel(x, e, e_subgraph, edge_index, edge_index_sub, W_node, b_node,
           W_edge, b_edge, A, bA, B, bB, C, bC, U, bU, V, bV, W_pred, b_pred):
    del e_subgraph
    dst = edge_index[1]
    src = edge_index[0]
    ssrc = edge_index_sub[0]
    sdst = edge_index_sub[1]

    bn = b_node.reshape(1, H)
    be = b_edge.reshape(1, H)
    cs = [jnp.moveaxis(C[i].reshape(H, 2, HH), 1, 0) for i in range(2)]
    bcs = [bC[i].reshape(2, 1, HH) for i in range(2)]
    w12 = jnp.stack([W_pred[:H, 0], W_pred[H:, 0]], axis=1)
    bvec = jnp.concatenate([b_pred, jnp.zeros((1,), _f32)]).reshape(1, 2)

    t0, uh0 = _node_first(
        x, W_node, bn, A[0], bA[0].reshape(1, H), B[0], bB[0].reshape(1, H),
        V[0], bV[0].reshape(1, H), U[0], bU[0].reshape(1, H))
    ce0 = _edge_first(e, W_edge, be, cs[0], bcs[0])
    nd0, ee1 = _edge_pass_l0(
        dst, src, t0, ce0.reshape(2 * E, HH))

    t1, uh1 = _node_next(
        uh0, nd0, A[1], bA[1].reshape(1, H), B[1], bB[1].reshape(1, H),
        V[1], bV[1].reshape(1, H), U[1], bU[1].reshape(1, H))
    ce1 = _edge_next(ee1.reshape(2, E, HH), cs[1], bcs[1])
    nd1 = _edge_pass_l1(
        dst, src, t1, ce1.reshape(2 * E, HH))

    p12 = _node_last(uh1, nd1, w12, bvec)
    scores = _score_pass(p12[:, 0], p12[:, 1], ssrc, sdst)
    return scores.reshape(E, 1)


# speedup vs baseline: 1.5212x; 1.1183x over previous
import functools

import jax
import jax.numpy as jnp
from jax import lax
from jax.experimental import pallas as pl
from jax.experimental.pallas import tpu as pltpu
from jax.experimental.pallas import tpu_sc as plsc

N = 10000
E = 320000
D_NODE = 128
D_EDGE = 16
H = 128
HH = 64
LN = 16
NC = 2
NS = 16
EPT = E // NS
CH = 40
NCHUNK = EPT // CH
NP = 10112
NPT = NP // NS
EPW = E // (NC * NS)
CH2 = 400
NCHUNK2 = EPW // CH2
BE = 4000

_f32 = jnp.float32


def _node_first_body(x_ref, wn_ref, bn_ref, a_ref, ba_ref, b_ref, bb_ref,
                     v_ref, bv_ref, u_ref, bu_ref, t_o, uh_o):
    h = jnp.dot(x_ref[...], wn_ref[...], preferred_element_type=_f32) + bn_ref[...]
    ah = jnp.dot(h, a_ref[...], preferred_element_type=_f32) + ba_ref[...]
    bh = jnp.dot(h, b_ref[...], preferred_element_type=_f32) + bb_ref[...]
    vh = jnp.dot(h, v_ref[...], preferred_element_type=_f32) + bv_ref[...]
    uh = jnp.dot(h, u_ref[...], preferred_element_type=_f32) + bu_ref[...]
    t_o[0 * N:1 * N] = ah
    t_o[1 * N:2 * N] = jnp.concatenate([bh[:, :HH], vh[:, :HH]], axis=1)
    t_o[2 * N:3 * N] = ah
    t_o[3 * N:4 * N] = jnp.concatenate([bh[:, HH:], vh[:, HH:]], axis=1)
    uh_o[...] = uh


def _node_next_body(uh_ref, nd_ref, a_ref, ba_ref, b_ref, bb_ref,
                    v_ref, bv_ref, u_ref, bu_ref, t_o, uh_o):
    nd = nd_ref[...]
    agg = jnp.concatenate(
        [nd[:N, :HH] / (nd[:N, HH:] + 1e-6),
         nd[NP:NP + N, :HH] / (nd[NP:NP + N, HH:] + 1e-6)], axis=1)
    h = jnp.maximum(uh_ref[...] + agg, 0.0)
    ah = jnp.dot(h, a_ref[...], preferred_element_type=_f32) + ba_ref[...]
    bh = jnp.dot(h, b_ref[...], preferred_element_type=_f32) + bb_ref[...]
    vh = jnp.dot(h, v_ref[...], preferred_element_type=_f32) + bv_ref[...]
    uh = jnp.dot(h, u_ref[...], preferred_element_type=_f32) + bu_ref[...]
    t_o[0 * N:1 * N] = ah
    t_o[1 * N:2 * N] = jnp.concatenate([bh[:, :HH], vh[:, :HH]], axis=1)
    t_o[2 * N:3 * N] = ah
    t_o[3 * N:4 * N] = jnp.concatenate([bh[:, HH:], vh[:, HH:]], axis=1)
    uh_o[...] = uh


def _node_last_body(uh_ref, nd_ref, w12_ref, bvec_ref, p_o):
    nd = nd_ref[...]
    agg = jnp.concatenate(
        [nd[:N, :HH] / (nd[:N, HH:] + 1e-6),
         nd[NP:NP + N, :HH] / (nd[NP:NP + N, HH:] + 1e-6)], axis=1)
    h = jnp.maximum(uh_ref[...] + agg, 0.0)
    p_o[...] = jnp.dot(h, w12_ref[...], preferred_element_type=_f32) + bvec_ref[...]


def _node_first(x, wn, bn, a, ba, b, bb, v, bv, u, bu):
    return pl.pallas_call(
        _node_first_body,
        out_shape=[
            jax.ShapeDtypeStruct((4 * N, H), _f32),
            jax.ShapeDtypeStruct((N, H), _f32),
        ],
    )(x, wn, bn, a, ba, b, bb, v, bv, u, bu)


def _node_next(uh, nd, a, ba, b, bb, v, bv, u, bu):
    return pl.pallas_call(
        _node_next_body,
        out_shape=[
            jax.ShapeDtypeStruct((4 * N, H), _f32),
            jax.ShapeDtypeStruct((N, H), _f32),
        ],
    )(uh, nd, a, ba, b, bb, v, bv, u, bu)


def _node_last(uh, nd, w12, bvec):
    return pl.pallas_call(
        _node_last_body,
        out_shape=jax.ShapeDtypeStruct((N, 2), _f32),
    )(uh, nd, w12, bvec)


def _edge_first_body(e_ref, we_ref, be_ref, cs_ref, bcs_ref, out_ref):
    ee = jnp.dot(e_ref[...], we_ref[...], preferred_element_type=_f32) + be_ref[...]
    out_ref[0] = (jnp.dot(ee, cs_ref[0], preferred_element_type=_f32)
                  + bcs_ref[0])


def _edge_first(e, we, be, cs, bcs):
    grid = (2, E // BE)
    return pl.pallas_call(
        _edge_first_body,
        grid=grid,
        in_specs=[
            pl.BlockSpec((BE, D_EDGE), lambda c, i: (i, 0)),
            pl.BlockSpec((D_EDGE, H), lambda c, i: (0, 0)),
            pl.BlockSpec((1, H), lambda c, i: (0, 0)),
            pl.BlockSpec((1, H, HH), lambda c, i: (c, 0, 0)),
            pl.BlockSpec((1, 1, HH), lambda c, i: (c, 0, 0)),
        ],
        out_specs=pl.BlockSpec((1, BE, HH), lambda c, i: (c, i, 0)),
        out_shape=jax.ShapeDtypeStruct((2, E, HH), _f32),
    )(e, we, be, cs, bcs)


def _edge_next_body(ee_ref, cs_ref, bcs_ref, out_ref):
    cblk = cs_ref[0]
    out_ref[0] = (jnp.dot(ee_ref[0], cblk[:HH], preferred_element_type=_f32)
                  + jnp.dot(ee_ref[1], cblk[HH:], preferred_element_type=_f32)
                  + bcs_ref[0])


def _edge_next(ee, cs, bcs):
    grid = (2, E // BE)
    return pl.pallas_call(
        _edge_next_body,
        grid=grid,
        in_specs=[
            pl.BlockSpec((2, BE, HH), lambda c, i: (0, i, 0)),
            pl.BlockSpec((1, H, HH), lambda c, i: (c, 0, 0)),
            pl.BlockSpec((1, 1, HH), lambda c, i: (c, 0, 0)),
        ],
        out_specs=pl.BlockSpec((1, BE, HH), lambda c, i: (c, i, 0)),
        out_shape=jax.ShapeDtypeStruct((2, E, HH), _f32),
    )(ee, cs, bcs)


def _make_edge_pass(write_ee):
    mesh = plsc.VectorSubcoreMesh(
        core_axis_name="c", subcore_axis_name="s", num_cores=NC, num_subcores=NS)
    if write_ee:
        out_type = [jax.ShapeDtypeStruct((2 * NP, H), _f32),
                    jax.ShapeDtypeStruct((2 * E, HH), _f32)]
    else:
        out_type = jax.ShapeDtypeStruct((2 * NP, H), _f32)
    scratch = [
        pltpu.VMEM((2, 2 * CH), jnp.int32),
        pltpu.VMEM((CH,), jnp.int32),
        pltpu.VMEM((CH,), jnp.int32),
        pltpu.VMEM((CH,), jnp.int32),
        pltpu.VMEM((CH, H), _f32),
        pltpu.VMEM((CH, H), _f32),
        pltpu.VMEM((CH, HH), _f32),
        pltpu.VMEM((CH, HH), _f32),
        pltpu.VMEM((CH, H), _f32),
        pltpu.VMEM_SHARED((NP, H), _f32),
        pltpu.SemaphoreType.DMA,
        pltpu.SemaphoreType.DMA,
        pltpu.SemaphoreType.DMA,
    ]

    @functools.partial(pl.kernel, out_type=out_type, mesh=mesh,
                       scratch_types=scratch)
    def edge_pass(idxc_hbm, t_hbm, ce_hbm, *rest):
        if write_ee:
            nd_out, ee_out = rest[0], rest[1]
            rest = rest[2:]
        else:
            nd_out = rest[0]
            ee_out = None
            rest = rest[1:]
        (dsrc_v, adst_v, bsrc_v, dsc_v, a_v, bv_v, ce_v, ee_v,
         ps_v, nd_sp, sem_i, sem_g, sem_w) = rest
        c = lax.axis_index("c")
        s = lax.axis_index("s")
        r0 = s * NPT
        def _z(i, carry):
            for kk in range(H // LN):
                ps_v[i, pl.ds(kk * LN, LN)] = jnp.zeros((LN,), _f32)
            return carry

        lax.fori_loop(0, 8, _z, 0)

        def _zcopy(k, carry):
            pltpu.sync_copy(ps_v.at[pl.ds(0, 8)],
                            nd_sp.at[pl.ds(r0 + k * 8, 8)])
            return carry

        lax.fori_loop(0, NPT // 8, _zcopy, 0)
        plsc.subcore_barrier()

        ebase = s * EPT
        cebase = c * E + ebase
        coff = c * (2 * N)

        ebase2 = 2 * s * EPT
        cebase = c * E + s * EPT
        coff = c * (2 * N)

        def issue_idx(g, b):
            pltpu.async_copy(idxc_hbm.at[pl.ds(ebase2 + g * 2 * CH, 2 * CH)],
                             dsrc_v.at[b], sem_i)

        def wait_idx(b):
            pltpu.make_async_copy(idxc_hbm.at[pl.ds(ebase2, 2 * CH)],
                                  dsrc_v.at[b], sem_i).wait()

        _WIN = (0, 16, CH - LN)

        def transform(b):
            for w in _WIN:
                sl = pl.ds(w, LN)
                d = dsrc_v[b, sl]
                adst_v[sl] = d + coff
                dsc_v[sl] = d
                bsrc_v[sl] = dsrc_v[b, pl.ds(CH + w, LN)] + (coff + N)

        abase = c * HH

        def compute():
            def edge(i, ecarry):
                for r in range(HH // LN):
                    sl = pl.ds(r * LN, LN)
                    sv = pl.ds(HH + r * LN, LN)
                    en = (a_v[i, pl.ds(abase + r * LN, LN)]
                          + bv_v[i, sl] + ce_v[i, sl])
                    sg = 1.0 / (1.0 + jnp.exp(-en))
                    ps_v[i, sl] = sg * bv_v[i, sv]
                    ps_v[i, sv] = sg
                    if write_ee:
                        ee_v[i, sl] = jnp.maximum(en, 0.0)
                return ecarry

            lax.fori_loop(0, CH, edge, 0)

        def wait_ee(g):
            if write_ee:
                pltpu.make_async_copy(ee_v, ee_out.at[pl.ds(cebase, CH)],
                                      sem_w).wait()

        def body(g, b, first, last):
            wait_idx(b)
            transform(b)
            g1 = pltpu.async_copy(t_hbm.at[adst_v], a_v, sem_g)
            g2 = pltpu.async_copy(t_hbm.at[bsrc_v], bv_v, sem_g)
            g3 = pltpu.async_copy(ce_hbm.at[pl.ds(cebase + g * CH, CH)],
                                  ce_v, sem_g)
            if not last:
                issue_idx(g + 1, 1 - b)
            if not first:
                wait_ee(g)
            g1.wait()
            g2.wait()
            g3.wait()
            compute()
            if write_ee:
                pltpu.async_copy(ee_v, ee_out.at[pl.ds(cebase + g * CH, CH)],
                                 sem_w)
            pltpu.sync_copy(ps_v, nd_sp.at[dsc_v], add=True)

        issue_idx(0, 0)
        body(0, 0, True, False)

        def chunk(jj, carry):
            for b in (1, 0):
                g = 2 * jj + (1 if b == 1 else 2)
                body(g, b, False, False)
            return carry

        lax.fori_loop(0, (NCHUNK - 2) // 2, chunk, 0)
        body(NCHUNK - 1, 1, False, True)
        if write_ee:
            pltpu.make_async_copy(ee_v, ee_out.at[pl.ds(cebase, CH)],
                                  sem_w).wait()
        plsc.subcore_barrier()
        pltpu.sync_copy(nd_sp.at[pl.ds(r0, NPT)],
                        nd_out.at[pl.ds(c * NP + r0, NPT)])

    return edge_pass


_edge_pass_l0 = _make_edge_pass(True)
_edge_pass_l1 = _make_edge_pass(False)


def _make_score():
    mesh = plsc.VectorSubcoreMesh(
        core_axis_name="c", subcore_axis_name="s", num_cores=NC, num_subcores=NS)
    scratch = [
        pltpu.VMEM((N,), _f32),
        pltpu.VMEM((N,), _f32),
        pltpu.VMEM((CH2,), jnp.int32),
        pltpu.VMEM((CH2,), jnp.int32),
        pltpu.VMEM((CH2,), _f32),
    ]

    @functools.partial(pl.kernel,
                       out_type=jax.ShapeDtypeStruct((E,), _f32),
                       mesh=mesh, scratch_types=scratch,
                       compiler_params=pltpu.CompilerParams(
                           needs_layout_passes=False))
    def score(p1_hbm, p2_hbm, ssrc_hbm, sdst_hbm, out_hbm,
              p1_v, p2_v, si_v, di_v, o_v):
        c = lax.axis_index("c")
        s = lax.axis_index("s")
        wid = s * NC + c
        pltpu.sync_copy(p1_hbm, p1_v)
        pltpu.sync_copy(p2_hbm, p2_v)
        wbase = wid * EPW

        def chunk(j, carry):
            base = wbase + j * CH2
            pltpu.sync_copy(ssrc_hbm.at[pl.ds(base, CH2)], si_v)
            pltpu.sync_copy(sdst_hbm.at[pl.ds(base, CH2)], di_v)

            def vec(kk, vcarry):
                sl = pl.ds(kk * LN, LN)
                g1 = plsc.load_gather(p1_v, [si_v[sl]])
                g2 = plsc.load_gather(p2_v, [di_v[sl]])
                o_v[sl] = g1 + g2
                return vcarry

            lax.fori_loop(0, CH2 // LN, vec, 0)
            pltpu.sync_copy(o_v, out_hbm.at[pl.ds(base, CH2)])
            return carry

        lax.fori_loop(0, NCHUNK2, chunk, 0)

    return score


_score_pass = _make_score()


def kernel(x, e, e_subgraph, edge_index, edge_index_sub, W_node, b_node,
           W_edge, b_edge, A, bA, B, bB, C, bC, U, bU, V, bV, W_pred, b_pred):
    del e_subgraph
    idxc = jnp.concatenate(
        [edge_index[1].reshape(NS, NCHUNK, 1, CH),
         edge_index[0].reshape(NS, NCHUNK, 1, CH)],
        axis=2).reshape(2 * E)
    ssrc = edge_index_sub[0]
    sdst = edge_index_sub[1]

    bn = b_node.reshape(1, H)
    be = b_edge.reshape(1, H)
    cs = [jnp.moveaxis(C[i].reshape(H, 2, HH), 1, 0) for i in range(2)]
    bcs = [bC[i].reshape(2, 1, HH) for i in range(2)]
    w12 = jnp.stack([W_pred[:H, 0], W_pred[H:, 0]], axis=1)
    bvec = jnp.concatenate([b_pred, jnp.zeros((1,), _f32)]).reshape(1, 2)

    t0, uh0 = _node_first(
        x, W_node, bn, A[0], bA[0].reshape(1, H), B[0], bB[0].reshape(1, H),
        V[0], bV[0].reshape(1, H), U[0], bU[0].reshape(1, H))
    ce0 = _edge_first(e, W_edge, be, cs[0], bcs[0])
    nd0, ee1 = _edge_pass_l0(
        idxc, t0, ce0.reshape(2 * E, HH))

    t1, uh1 = _node_next(
        uh0, nd0, A[1], bA[1].reshape(1, H), B[1], bB[1].reshape(1, H),
        V[1], bV[1].reshape(1, H), U[1], bU[1].reshape(1, H))
    ce1 = _edge_next(ee1.reshape(2, E, HH), cs[1], bcs[1])
    nd1 = _edge_pass_l1(
        idxc, t1, ce1.reshape(2 * E, HH))

    p12 = _node_last(uh1, nd1, w12, bvec)
    scores = _score_pass(p12[:, 0], p12[:, 1], ssrc, sdst)
    return scores.reshape(E, 1)


# speedup vs baseline: 1.9375x; 1.2737x over previous
import functools

import jax
import jax.numpy as jnp
from jax import lax
from jax.experimental import pallas as pl
from jax.experimental.pallas import tpu as pltpu
from jax.experimental.pallas import tpu_sc as plsc

N = 10000
E = 320000
D_NODE = 128
D_EDGE = 16
H = 128
HH = 64
LN = 16
NC = 2
NS = 16
EPT = E // NS
CH = 40
NCHUNK = EPT // CH
NP = 10112
NPT = NP // NS
EPW = E // (NC * NS)
CH2 = 400
NCHUNK2 = EPW // CH2
BE = 4000

_f32 = jnp.float32


def _node_first_body(x_ref, wn_ref, bn_ref, a_ref, ba_ref, b_ref, bb_ref,
                     v_ref, bv_ref, u_ref, bu_ref, t_o, uh_o):
    h = jnp.dot(x_ref[...], wn_ref[...], preferred_element_type=_f32) + bn_ref[...]
    ah = jnp.dot(h, a_ref[...], preferred_element_type=_f32) + ba_ref[...]
    bh = jnp.dot(h, b_ref[...], preferred_element_type=_f32) + bb_ref[...]
    vh = jnp.dot(h, v_ref[...], preferred_element_type=_f32) + bv_ref[...]
    uh = jnp.dot(h, u_ref[...], preferred_element_type=_f32) + bu_ref[...]
    t_o[0 * N:1 * N] = ah
    t_o[1 * N:2 * N] = jnp.concatenate([bh[:, :HH], vh[:, :HH]], axis=1)
    t_o[2 * N:3 * N] = ah
    t_o[3 * N:4 * N] = jnp.concatenate([bh[:, HH:], vh[:, HH:]], axis=1)
    uh_o[...] = uh


def _node_next_body(uh_ref, nd_ref, a_ref, ba_ref, b_ref, bb_ref,
                    v_ref, bv_ref, u_ref, bu_ref, t_o, uh_o):
    nd = nd_ref[...]
    agg = jnp.concatenate(
        [nd[:N, :HH] / (nd[:N, HH:] + 1e-6),
         nd[NP:NP + N, :HH] / (nd[NP:NP + N, HH:] + 1e-6)], axis=1)
    h = jnp.maximum(uh_ref[...] + agg, 0.0)
    ah = jnp.dot(h, a_ref[...], preferred_element_type=_f32) + ba_ref[...]
    bh = jnp.dot(h, b_ref[...], preferred_element_type=_f32) + bb_ref[...]
    vh = jnp.dot(h, v_ref[...], preferred_element_type=_f32) + bv_ref[...]
    uh = jnp.dot(h, u_ref[...], preferred_element_type=_f32) + bu_ref[...]
    t_o[0 * N:1 * N] = ah
    t_o[1 * N:2 * N] = jnp.concatenate([bh[:, :HH], vh[:, :HH]], axis=1)
    t_o[2 * N:3 * N] = ah
    t_o[3 * N:4 * N] = jnp.concatenate([bh[:, HH:], vh[:, HH:]], axis=1)
    uh_o[...] = uh


def _node_last_body(uh_ref, nd_ref, w12_ref, bvec_ref, p_o):
    nd = nd_ref[...]
    agg = jnp.concatenate(
        [nd[:N, :HH] / (nd[:N, HH:] + 1e-6),
         nd[NP:NP + N, :HH] / (nd[NP:NP + N, HH:] + 1e-6)], axis=1)
    h = jnp.maximum(uh_ref[...] + agg, 0.0)
    p_o[...] = jnp.dot(h, w12_ref[...], preferred_element_type=_f32) + bvec_ref[...]


def _node_first(x, wn, bn, a, ba, b, bb, v, bv, u, bu):
    return pl.pallas_call(
        _node_first_body,
        out_shape=[
            jax.ShapeDtypeStruct((4 * N, H), _f32),
            jax.ShapeDtypeStruct((N, H), _f32),
        ],
    )(x, wn, bn, a, ba, b, bb, v, bv, u, bu)


def _node_next(uh, nd, a, ba, b, bb, v, bv, u, bu):
    return pl.pallas_call(
        _node_next_body,
        out_shape=[
            jax.ShapeDtypeStruct((4 * N, H), _f32),
            jax.ShapeDtypeStruct((N, H), _f32),
        ],
    )(uh, nd, a, ba, b, bb, v, bv, u, bu)


def _node_last(uh, nd, w12, bvec):
    return pl.pallas_call(
        _node_last_body,
        out_shape=jax.ShapeDtypeStruct((N, 2), _f32),
    )(uh, nd, w12, bvec)


def _edge_first_body(e_ref, we_ref, be_ref, cs_ref, bcs_ref, out_ref):
    ee = jnp.dot(e_ref[...], we_ref[...], preferred_element_type=_f32) + be_ref[...]
    out_ref[0] = (jnp.dot(ee, cs_ref[0], preferred_element_type=_f32)
                  + bcs_ref[0])


def _edge_first(e, we, be, cs, bcs):
    grid = (2, E // BE)
    return pl.pallas_call(
        _edge_first_body,
        grid=grid,
        in_specs=[
            pl.BlockSpec((BE, D_EDGE), lambda c, i: (i, 0)),
            pl.BlockSpec((D_EDGE, H), lambda c, i: (0, 0)),
            pl.BlockSpec((1, H), lambda c, i: (0, 0)),
            pl.BlockSpec((1, H, HH), lambda c, i: (c, 0, 0)),
            pl.BlockSpec((1, 1, HH), lambda c, i: (c, 0, 0)),
        ],
        out_specs=pl.BlockSpec((1, BE, HH), lambda c, i: (c, i, 0)),
        out_shape=jax.ShapeDtypeStruct((2, E, HH), _f32),
    )(e, we, be, cs, bcs)


def _edge_next_body(ee_ref, cs_ref, bcs_ref, out_ref):
    cblk = cs_ref[0]
    out_ref[0] = (jnp.dot(ee_ref[0], cblk[:HH], preferred_element_type=_f32)
                  + jnp.dot(ee_ref[1], cblk[HH:], preferred_element_type=_f32)
                  + bcs_ref[0])


def _edge_next(ee, cs, bcs):
    grid = (2, E // BE)
    return pl.pallas_call(
        _edge_next_body,
        grid=grid,
        in_specs=[
            pl.BlockSpec((2, BE, HH), lambda c, i: (0, i, 0)),
            pl.BlockSpec((1, H, HH), lambda c, i: (c, 0, 0)),
            pl.BlockSpec((1, 1, HH), lambda c, i: (c, 0, 0)),
        ],
        out_specs=pl.BlockSpec((1, BE, HH), lambda c, i: (c, i, 0)),
        out_shape=jax.ShapeDtypeStruct((2, E, HH), _f32),
    )(ee, cs, bcs)


def _make_edge_pass(write_ee):
    mesh = plsc.VectorSubcoreMesh(
        core_axis_name="c", subcore_axis_name="s", num_cores=NC, num_subcores=NS)
    if write_ee:
        out_type = [jax.ShapeDtypeStruct((2 * NP, H), _f32),
                    jax.ShapeDtypeStruct((2 * E, HH), _f32)]
    else:
        out_type = jax.ShapeDtypeStruct((2 * NP, H), _f32)
    scratch = [
        pltpu.VMEM((2, 2 * CH), jnp.int32),
        pltpu.VMEM((2, CH), jnp.int32),
        pltpu.VMEM((2, CH), jnp.int32),
        pltpu.VMEM((2, CH), jnp.int32),
        pltpu.VMEM((2, CH, H), _f32),
        pltpu.VMEM((2, CH, H), _f32),
        pltpu.VMEM((2, CH, HH), _f32),
        pltpu.VMEM((CH, HH), _f32),
        pltpu.VMEM((CH, H), _f32),
        pltpu.VMEM_SHARED((NP, H), _f32),
        pltpu.SemaphoreType.DMA,
        pltpu.SemaphoreType.DMA,
        pltpu.SemaphoreType.DMA,
    ]

    @functools.partial(pl.kernel, out_type=out_type, mesh=mesh,
                       scratch_types=scratch)
    def edge_pass(idxc_hbm, t_hbm, ce_hbm, *rest):
        if write_ee:
            nd_out, ee_out = rest[0], rest[1]
            rest = rest[2:]
        else:
            nd_out = rest[0]
            ee_out = None
            rest = rest[1:]
        (dsrc_v, adst_v, bsrc_v, dsc_v, a_v, bv_v, ce_v, ee_v,
         ps_v, nd_sp, sem_i, sem_g, sem_w) = rest
        c = lax.axis_index("c")
        s = lax.axis_index("s")
        r0 = s * NPT
        def _z(i, carry):
            for kk in range(H // LN):
                ps_v[i, pl.ds(kk * LN, LN)] = jnp.zeros((LN,), _f32)
            return carry

        lax.fori_loop(0, 8, _z, 0)

        def _zcopy(k, carry):
            pltpu.sync_copy(ps_v.at[pl.ds(0, 8)],
                            nd_sp.at[pl.ds(r0 + k * 8, 8)])
            return carry

        lax.fori_loop(0, NPT // 8, _zcopy, 0)
        plsc.subcore_barrier()

        ebase = s * EPT
        cebase = c * E + ebase
        coff = c * (2 * N)

        ebase2 = 2 * s * EPT
        cebase = c * E + s * EPT
        coff = c * (2 * N)

        def issue_idx(g, b):
            pltpu.async_copy(idxc_hbm.at[pl.ds(ebase2 + g * 2 * CH, 2 * CH)],
                             dsrc_v.at[b], sem_i)

        def wait_idx(b):
            pltpu.make_async_copy(idxc_hbm.at[pl.ds(ebase2, 2 * CH)],
                                  dsrc_v.at[b], sem_i).wait()

        _WIN = (0, 16, CH - LN)

        def transform(b):
            for w in _WIN:
                sl = pl.ds(w, LN)
                d = dsrc_v[b, sl]
                adst_v[b, sl] = d + coff
                dsc_v[b, sl] = d
                bsrc_v[b, sl] = dsrc_v[b, pl.ds(CH + w, LN)] + (coff + N)

        def issue_gathers(g, b):
            return (
                pltpu.async_copy(t_hbm.at[adst_v.at[b]], a_v.at[b], sem_g),
                pltpu.async_copy(t_hbm.at[bsrc_v.at[b]], bv_v.at[b], sem_g),
                pltpu.async_copy(ce_hbm.at[pl.ds(cebase + g * CH, CH)],
                                 ce_v.at[b], sem_g),
            )

        def wait_gathers(b):
            pltpu.make_async_copy(t_hbm.at[adst_v.at[b]], a_v.at[b],
                                  sem_g).wait()
            pltpu.make_async_copy(t_hbm.at[bsrc_v.at[b]], bv_v.at[b],
                                  sem_g).wait()
            pltpu.make_async_copy(ce_hbm.at[pl.ds(cebase, CH)], ce_v.at[b],
                                  sem_g).wait()

        abase = c * HH

        def compute(b):
            def edge(i, ecarry):
                for r in range(HH // LN):
                    sl = pl.ds(r * LN, LN)
                    sv = pl.ds(HH + r * LN, LN)
                    en = (a_v[b, i, pl.ds(abase + r * LN, LN)]
                          + bv_v[b, i, sl] + ce_v[b, i, sl])
                    sg = 1.0 / (1.0 + jnp.exp(-en))
                    ps_v[i, sl] = sg * bv_v[b, i, sv]
                    ps_v[i, sv] = sg
                    if write_ee:
                        ee_v[i, sl] = jnp.maximum(en, 0.0)
                return ecarry

            lax.fori_loop(0, CH, edge, 0)

        def wait_ee():
            if write_ee:
                pltpu.make_async_copy(ee_v, ee_out.at[pl.ds(cebase, CH)],
                                      sem_w).wait()

        def body(g, b, first, last):
            if not last:
                wait_idx(1 - b)
                transform(1 - b)
            wait_gathers(b)
            if not last:
                issue_gathers(g + 1, 1 - b)
            issue_idx(jnp.minimum(g + 2, NCHUNK - 1), b)
            if not first:
                wait_ee()
            compute(b)
            if write_ee:
                pltpu.async_copy(ee_v, ee_out.at[pl.ds(cebase + g * CH, CH)],
                                 sem_w)
            pltpu.sync_copy(ps_v, nd_sp.at[dsc_v.at[b]], add=True)

        issue_idx(0, 0)
        wait_idx(0)
        transform(0)
        issue_gathers(0, 0)
        issue_idx(1, 1)
        body(0, 0, True, False)

        def chunk(jj, carry):
            for b in (1, 0):
                g = 2 * jj + (1 if b == 1 else 2)
                body(g, b, False, False)
            return carry

        lax.fori_loop(0, (NCHUNK - 2) // 2, chunk, 0)
        body(NCHUNK - 1, 1, False, True)
        wait_ee()
        wait_idx(0)
        wait_idx(1)
        plsc.subcore_barrier()
        pltpu.sync_copy(nd_sp.at[pl.ds(r0, NPT)],
                        nd_out.at[pl.ds(c * NP + r0, NPT)])

    return edge_pass


_edge_pass_l0 = _make_edge_pass(True)
_edge_pass_l1 = _make_edge_pass(False)


def _make_score():
    mesh = plsc.VectorSubcoreMesh(
        core_axis_name="c", subcore_axis_name="s", num_cores=NC, num_subcores=NS)
    scratch = [
        pltpu.VMEM((N,), _f32),
        pltpu.VMEM((N,), _f32),
        pltpu.VMEM((CH2,), jnp.int32),
        pltpu.VMEM((CH2,), jnp.int32),
        pltpu.VMEM((CH2,), _f32),
    ]

    @functools.partial(pl.kernel,
                       out_type=jax.ShapeDtypeStruct((E,), _f32),
                       mesh=mesh, scratch_types=scratch,
                       compiler_params=pltpu.CompilerParams(
                           needs_layout_passes=False))
    def score(p1_hbm, p2_hbm, ssrc_hbm, sdst_hbm, out_hbm,
              p1_v, p2_v, si_v, di_v, o_v):
        c = lax.axis_index("c")
        s = lax.axis_index("s")
        wid = s * NC + c
        pltpu.sync_copy(p1_hbm, p1_v)
        pltpu.sync_copy(p2_hbm, p2_v)
        wbase = wid * EPW

        def chunk(j, carry):
            base = wbase + j * CH2
            pltpu.sync_copy(ssrc_hbm.at[pl.ds(base, CH2)], si_v)
            pltpu.sync_copy(sdst_hbm.at[pl.ds(base, CH2)], di_v)

            def vec(kk, vcarry):
                sl = pl.ds(kk * LN, LN)
                g1 = plsc.load_gather(p1_v, [si_v[sl]])
                g2 = plsc.load_gather(p2_v, [di_v[sl]])
                o_v[sl] = g1 + g2
                return vcarry

            lax.fori_loop(0, CH2 // LN, vec, 0)
            pltpu.sync_copy(o_v, out_hbm.at[pl.ds(base, CH2)])
            return carry

        lax.fori_loop(0, NCHUNK2, chunk, 0)

    return score


_score_pass = _make_score()


def kernel(x, e, e_subgraph, edge_index, edge_index_sub, W_node, b_node,
           W_edge, b_edge, A, bA, B, bB, C, bC, U, bU, V, bV, W_pred, b_pred):
    del e_subgraph
    idxc = jnp.concatenate(
        [edge_index[1].reshape(NS, NCHUNK, 1, CH),
         edge_index[0].reshape(NS, NCHUNK, 1, CH)],
        axis=2).reshape(2 * E)
    ssrc = edge_index_sub[0]
    sdst = edge_index_sub[1]

    bn = b_node.reshape(1, H)
    be = b_edge.reshape(1, H)
    cs = [jnp.moveaxis(C[i].reshape(H, 2, HH), 1, 0) for i in range(2)]
    bcs = [bC[i].reshape(2, 1, HH) for i in range(2)]
    w12 = jnp.stack([W_pred[:H, 0], W_pred[H:, 0]], axis=1)
    bvec = jnp.concatenate([b_pred, jnp.zeros((1,), _f32)]).reshape(1, 2)

    t0, uh0 = _node_first(
        x, W_node, bn, A[0], bA[0].reshape(1, H), B[0], bB[0].reshape(1, H),
        V[0], bV[0].reshape(1, H), U[0], bU[0].reshape(1, H))
    ce0 = _edge_first(e, W_edge, be, cs[0], bcs[0])
    nd0, ee1 = _edge_pass_l0(
        idxc, t0, ce0.reshape(2 * E, HH))

    t1, uh1 = _node_next(
        uh0, nd0, A[1], bA[1].reshape(1, H), B[1], bB[1].reshape(1, H),
        V[1], bV[1].reshape(1, H), U[1], bU[1].reshape(1, H))
    ce1 = _edge_next(ee1.reshape(2, E, HH), cs[1], bcs[1])
    nd1 = _edge_pass_l1(
        idxc, t1, ce1.reshape(2 * E, HH))

    p12 = _node_last(uh1, nd1, w12, bvec)
    scores = _score_pass(p12[:, 0], p12[:, 1], ssrc, sdst)
    return scores.reshape(E, 1)


# speedup vs baseline: 1.9774x; 1.0205x over previous
import functools

import jax
import jax.numpy as jnp
from jax import lax
from jax.experimental import pallas as pl
from jax.experimental.pallas import tpu as pltpu
from jax.experimental.pallas import tpu_sc as plsc

N = 10000
E = 320000
D_NODE = 128
D_EDGE = 16
H = 128
HH = 64
LN = 16
NC = 2
NS = 16
EPT = E // NS
CH = 40
NCHUNK = EPT // CH
NP = 10112
NPT = NP // NS
EPW = E // (NC * NS)
CH2 = 400
NCHUNK2 = EPW // CH2
BE = 4000

_f32 = jnp.float32


def _node_first_body(x_ref, wn_ref, bn_ref, a_ref, ba_ref, b_ref, bb_ref,
                     v_ref, bv_ref, u_ref, bu_ref, t_o, uh_o):
    h = jnp.dot(x_ref[...], wn_ref[...], preferred_element_type=_f32) + bn_ref[...]
    ah = jnp.dot(h, a_ref[...], preferred_element_type=_f32) + ba_ref[...]
    bh = jnp.dot(h, b_ref[...], preferred_element_type=_f32) + bb_ref[...]
    vh = jnp.dot(h, v_ref[...], preferred_element_type=_f32) + bv_ref[...]
    uh = jnp.dot(h, u_ref[...], preferred_element_type=_f32) + bu_ref[...]
    t_o[0 * N:1 * N] = ah
    t_o[1 * N:2 * N] = jnp.concatenate([bh[:, :HH], vh[:, :HH]], axis=1)
    t_o[2 * N:3 * N] = ah
    t_o[3 * N:4 * N] = jnp.concatenate([bh[:, HH:], vh[:, HH:]], axis=1)
    uh_o[...] = uh


def _node_next_body(uh_ref, nd_ref, a_ref, ba_ref, b_ref, bb_ref,
                    v_ref, bv_ref, u_ref, bu_ref, t_o, uh_o):
    nd = nd_ref[...]
    agg = jnp.concatenate(
        [nd[:N, :HH] / (nd[:N, HH:] + 1e-6),
         nd[NP:NP + N, :HH] / (nd[NP:NP + N, HH:] + 1e-6)], axis=1)
    h = jnp.maximum(uh_ref[...] + agg, 0.0)
    ah = jnp.dot(h, a_ref[...], preferred_element_type=_f32) + ba_ref[...]
    bh = jnp.dot(h, b_ref[...], preferred_element_type=_f32) + bb_ref[...]
    vh = jnp.dot(h, v_ref[...], preferred_element_type=_f32) + bv_ref[...]
    uh = jnp.dot(h, u_ref[...], preferred_element_type=_f32) + bu_ref[...]
    t_o[0 * N:1 * N] = ah
    t_o[1 * N:2 * N] = jnp.concatenate([bh[:, :HH], vh[:, :HH]], axis=1)
    t_o[2 * N:3 * N] = ah
    t_o[3 * N:4 * N] = jnp.concatenate([bh[:, HH:], vh[:, HH:]], axis=1)
    uh_o[...] = uh


def _node_last_body(uh_ref, nd_ref, w12_ref, bvec_ref, p_o):
    nd = nd_ref[...]
    agg = jnp.concatenate(
        [nd[:N, :HH] / (nd[:N, HH:] + 1e-6),
         nd[NP:NP + N, :HH] / (nd[NP:NP + N, HH:] + 1e-6)], axis=1)
    h = jnp.maximum(uh_ref[...] + agg, 0.0)
    p_o[...] = jnp.dot(h, w12_ref[...], preferred_element_type=_f32) + bvec_ref[...]


def _node_first(x, wn, bn, a, ba, b, bb, v, bv, u, bu):
    return pl.pallas_call(
        _node_first_body,
        out_shape=[
            jax.ShapeDtypeStruct((4 * N, H), _f32),
            jax.ShapeDtypeStruct((N, H), _f32),
        ],
    )(x, wn, bn, a, ba, b, bb, v, bv, u, bu)


def _node_next(uh, nd, a, ba, b, bb, v, bv, u, bu):
    return pl.pallas_call(
        _node_next_body,
        out_shape=[
            jax.ShapeDtypeStruct((4 * N, H), _f32),
            jax.ShapeDtypeStruct((N, H), _f32),
        ],
    )(uh, nd, a, ba, b, bb, v, bv, u, bu)


def _node_last(uh, nd, w12, bvec):
    return pl.pallas_call(
        _node_last_body,
        out_shape=jax.ShapeDtypeStruct((N, 2), _f32),
    )(uh, nd, w12, bvec)


def _edge_first_body(e_ref, we_ref, be_ref, cs_ref, bcs_ref, out_ref):
    ee = jnp.dot(e_ref[...], we_ref[...], preferred_element_type=_f32) + be_ref[...]
    out_ref[0] = (jnp.dot(ee, cs_ref[0], preferred_element_type=_f32)
                  + bcs_ref[0])


def _edge_first(e, we, be, cs, bcs):
    grid = (2, E // BE)
    return pl.pallas_call(
        _edge_first_body,
        grid=grid,
        in_specs=[
            pl.BlockSpec((BE, D_EDGE), lambda c, i: (i, 0)),
            pl.BlockSpec((D_EDGE, H), lambda c, i: (0, 0)),
            pl.BlockSpec((1, H), lambda c, i: (0, 0)),
            pl.BlockSpec((1, H, HH), lambda c, i: (c, 0, 0)),
            pl.BlockSpec((1, 1, HH), lambda c, i: (c, 0, 0)),
        ],
        out_specs=pl.BlockSpec((1, BE, HH), lambda c, i: (c, i, 0)),
        out_shape=jax.ShapeDtypeStruct((2, E, HH), _f32),
    )(e, we, be, cs, bcs)


def _edge_next_body(ee_ref, cs_ref, bcs_ref, out_ref):
    cblk = cs_ref[0]
    out_ref[0] = (jnp.dot(ee_ref[0], cblk[:HH], preferred_element_type=_f32)
                  + jnp.dot(ee_ref[1], cblk[HH:], preferred_element_type=_f32)
                  + bcs_ref[0])


def _edge_next(ee, cs, bcs):
    grid = (2, E // BE)
    return pl.pallas_call(
        _edge_next_body,
        grid=grid,
        in_specs=[
            pl.BlockSpec((2, BE, HH), lambda c, i: (0, i, 0)),
            pl.BlockSpec((1, H, HH), lambda c, i: (c, 0, 0)),
            pl.BlockSpec((1, 1, HH), lambda c, i: (c, 0, 0)),
        ],
        out_specs=pl.BlockSpec((1, BE, HH), lambda c, i: (c, i, 0)),
        out_shape=jax.ShapeDtypeStruct((2, E, HH), _f32),
    )(ee, cs, bcs)


def _make_edge_pass(write_ee):
    mesh = plsc.VectorSubcoreMesh(
        core_axis_name="c", subcore_axis_name="s", num_cores=NC, num_subcores=NS)
    if write_ee:
        out_type = [jax.ShapeDtypeStruct((2 * NP, H), _f32),
                    jax.ShapeDtypeStruct((2 * E, HH), _f32)]
    else:
        out_type = jax.ShapeDtypeStruct((2 * NP, H), _f32)
    scratch = [
        pltpu.VMEM((2, 2 * CH), jnp.int32),
        pltpu.VMEM((2, CH), jnp.int32),
        pltpu.VMEM((2, CH), jnp.int32),
        pltpu.VMEM((2, CH), jnp.int32),
        pltpu.VMEM((2, CH, H), _f32),
        pltpu.VMEM((2, CH, H), _f32),
        pltpu.VMEM((2, CH, HH), _f32),
        pltpu.VMEM((CH, HH), _f32),
        pltpu.VMEM((CH, H), _f32),
        pltpu.VMEM_SHARED((NP, H), _f32),
        pltpu.SemaphoreType.DMA,
        pltpu.SemaphoreType.DMA,
        pltpu.SemaphoreType.DMA,
    ]

    @functools.partial(pl.kernel, out_type=out_type, mesh=mesh,
                       scratch_types=scratch)
    def edge_pass(idxc_hbm, t_hbm, ce_hbm, *rest):
        if write_ee:
            nd_out, ee_out = rest[0], rest[1]
            rest = rest[2:]
        else:
            nd_out = rest[0]
            ee_out = None
            rest = rest[1:]
        (dsrc_v, adst_v, bsrc_v, dsc_v, a_v, bv_v, ce_v, ee_v,
         ps_v, nd_sp, sem_i, sem_g, sem_w) = rest
        c = lax.axis_index("c")
        s = lax.axis_index("s")
        r0 = s * NPT
        def _z(i, carry):
            for kk in range(H // LN):
                ps_v[i, pl.ds(kk * LN, LN)] = jnp.zeros((LN,), _f32)
            return carry

        lax.fori_loop(0, 8, _z, 0)

        def _zcopy(k, carry):
            pltpu.sync_copy(ps_v.at[pl.ds(0, 8)],
                            nd_sp.at[pl.ds(r0 + k * 8, 8)])
            return carry

        lax.fori_loop(0, NPT // 8, _zcopy, 0)
        plsc.subcore_barrier()

        ebase = s * EPT
        cebase = c * E + ebase
        coff = c * (2 * N)

        ebase2 = 2 * s * EPT
        cebase = c * E + s * EPT
        coff = c * (2 * N)

        def issue_idx(g, b):
            pltpu.async_copy(idxc_hbm.at[pl.ds(ebase2 + g * 2 * CH, 2 * CH)],
                             dsrc_v.at[b], sem_i)

        def wait_idx(b):
            pltpu.make_async_copy(idxc_hbm.at[pl.ds(ebase2, 2 * CH)],
                                  dsrc_v.at[b], sem_i).wait()

        _WIN = (0, 16, CH - LN)

        def transform(b):
            for w in _WIN:
                sl = pl.ds(w, LN)
                d = dsrc_v[b, sl]
                adst_v[b, sl] = d + coff
                dsc_v[b, sl] = d
                bsrc_v[b, sl] = dsrc_v[b, pl.ds(CH + w, LN)] + (coff + N)

        def issue_gathers(g, b):
            return (
                pltpu.async_copy(t_hbm.at[adst_v.at[b]], a_v.at[b], sem_g),
                pltpu.async_copy(t_hbm.at[bsrc_v.at[b]], bv_v.at[b], sem_g),
                pltpu.async_copy(ce_hbm.at[pl.ds(cebase + g * CH, CH)],
                                 ce_v.at[b], sem_g),
            )

        def wait_gathers(b):
            pltpu.make_async_copy(t_hbm.at[adst_v.at[b]], a_v.at[b],
                                  sem_g).wait()
            pltpu.make_async_copy(t_hbm.at[bsrc_v.at[b]], bv_v.at[b],
                                  sem_g).wait()
            pltpu.make_async_copy(ce_hbm.at[pl.ds(cebase, CH)], ce_v.at[b],
                                  sem_g).wait()

        abase = c * HH

        def compute(b):
            def edge(i4, ecarry):
                for u in range(4):
                    i = i4 * 4 + u
                    for r in range(HH // LN):
                        sl = pl.ds(r * LN, LN)
                        sv = pl.ds(HH + r * LN, LN)
                        en = (a_v[b, i, pl.ds(abase + r * LN, LN)]
                              + bv_v[b, i, sl] + ce_v[b, i, sl])
                        sg = 1.0 / (1.0 + jnp.exp(-en))
                        ps_v[i, sl] = sg * bv_v[b, i, sv]
                        ps_v[i, sv] = sg
                        if write_ee:
                            ee_v[i, sl] = jnp.maximum(en, 0.0)
                return ecarry

            lax.fori_loop(0, CH // 4, edge, 0)

        def wait_ee():
            if write_ee:
                pltpu.make_async_copy(ee_v, ee_out.at[pl.ds(cebase, CH)],
                                      sem_w).wait()

        def body(g, b, first, last):
            if not last:
                wait_idx(1 - b)
                transform(1 - b)
            wait_gathers(b)
            if not last:
                issue_gathers(g + 1, 1 - b)
            issue_idx(jnp.minimum(g + 2, NCHUNK - 1), b)
            if not first:
                wait_ee()
            compute(b)
            if write_ee:
                pltpu.async_copy(ee_v, ee_out.at[pl.ds(cebase + g * CH, CH)],
                                 sem_w)
            pltpu.sync_copy(ps_v, nd_sp.at[dsc_v.at[b]], add=True)

        issue_idx(0, 0)
        wait_idx(0)
        transform(0)
        issue_gathers(0, 0)
        issue_idx(1, 1)
        body(0, 0, True, False)

        def chunk(jj, carry):
            for b in (1, 0):
                g = 2 * jj + (1 if b == 1 else 2)
                body(g, b, False, False)
            return carry

        lax.fori_loop(0, (NCHUNK - 2) // 2, chunk, 0)
        body(NCHUNK - 1, 1, False, True)
        wait_ee()
        wait_idx(0)
        wait_idx(1)
        plsc.subcore_barrier()
        pltpu.sync_copy(nd_sp.at[pl.ds(r0, NPT)],
                        nd_out.at[pl.ds(c * NP + r0, NPT)])

    return edge_pass


_edge_pass_l0 = _make_edge_pass(True)
_edge_pass_l1 = _make_edge_pass(False)


def _make_score():
    mesh = plsc.VectorSubcoreMesh(
        core_axis_name="c", subcore_axis_name="s", num_cores=NC, num_subcores=NS)
    scratch = [
        pltpu.VMEM((N,), _f32),
        pltpu.VMEM((N,), _f32),
        pltpu.VMEM((CH2,), jnp.int32),
        pltpu.VMEM((CH2,), jnp.int32),
        pltpu.VMEM((CH2,), _f32),
    ]

    @functools.partial(pl.kernel,
                       out_type=jax.ShapeDtypeStruct((E,), _f32),
                       mesh=mesh, scratch_types=scratch,
                       compiler_params=pltpu.CompilerParams(
                           needs_layout_passes=False))
    def score(p1_hbm, p2_hbm, ssrc_hbm, sdst_hbm, out_hbm,
              p1_v, p2_v, si_v, di_v, o_v):
        c = lax.axis_index("c")
        s = lax.axis_index("s")
        wid = s * NC + c
        pltpu.sync_copy(p1_hbm, p1_v)
        pltpu.sync_copy(p2_hbm, p2_v)
        wbase = wid * EPW

        def chunk(j, carry):
            base = wbase + j * CH2
            pltpu.sync_copy(ssrc_hbm.at[pl.ds(base, CH2)], si_v)
            pltpu.sync_copy(sdst_hbm.at[pl.ds(base, CH2)], di_v)

            def vec(kk, vcarry):
                sl = pl.ds(kk * LN, LN)
                g1 = plsc.load_gather(p1_v, [si_v[sl]])
                g2 = plsc.load_gather(p2_v, [di_v[sl]])
                o_v[sl] = g1 + g2
                return vcarry

            lax.fori_loop(0, CH2 // LN, vec, 0)
            pltpu.sync_copy(o_v, out_hbm.at[pl.ds(base, CH2)])
            return carry

        lax.fori_loop(0, NCHUNK2, chunk, 0)

    return score


_score_pass = _make_score()


def kernel(x, e, e_subgraph, edge_index, edge_index_sub, W_node, b_node,
           W_edge, b_edge, A, bA, B, bB, C, bC, U, bU, V, bV, W_pred, b_pred):
    del e_subgraph
    idxc = jnp.concatenate(
        [edge_index[1].reshape(NS, NCHUNK, 1, CH),
         edge_index[0].reshape(NS, NCHUNK, 1, CH)],
        axis=2).reshape(2 * E)
    ssrc = edge_index_sub[0]
    sdst = edge_index_sub[1]

    bn = b_node.reshape(1, H)
    be = b_edge.reshape(1, H)
    cs = [jnp.moveaxis(C[i].reshape(H, 2, HH), 1, 0) for i in range(2)]
    bcs = [bC[i].reshape(2, 1, HH) for i in range(2)]
    w12 = jnp.stack([W_pred[:H, 0], W_pred[H:, 0]], axis=1)
    bvec = jnp.concatenate([b_pred, jnp.zeros((1,), _f32)]).reshape(1, 2)

    t0, uh0 = _node_first(
        x, W_node, bn, A[0], bA[0].reshape(1, H), B[0], bB[0].reshape(1, H),
        V[0], bV[0].reshape(1, H), U[0], bU[0].reshape(1, H))
    ce0 = _edge_first(e, W_edge, be, cs[0], bcs[0])
    nd0, ee1 = _edge_pass_l0(
        idxc, t0, ce0.reshape(2 * E, HH))

    t1, uh1 = _node_next(
        uh0, nd0, A[1], bA[1].reshape(1, H), B[1], bB[1].reshape(1, H),
        V[1], bV[1].reshape(1, H), U[1], bU[1].reshape(1, H))
    ce1 = _edge_next(ee1.reshape(2, E, HH), cs[1], bcs[1])
    nd1 = _edge_pass_l1(
        idxc, t1, ce1.reshape(2 * E, HH))

    p12 = _node_last(uh1, nd1, w12, bvec)
    scores = _score_pass(p12[:, 0], p12[:, 1], ssrc, sdst)
    return scores.reshape(E, 1)
